# baseline (device time: 193971 ns/iter reference)
import jax
import jax.numpy as jnp
from jax import lax
from jax.experimental import pallas as pl
from jax.experimental.pallas import tpu as pltpu

K = 32
N_CHUNKS = 4
NEG_INF = float("-inf")


def _topk_desc(x, k):
    cols = []
    for _ in range(k):
        m = jnp.max(x, axis=1, keepdims=True)
        cols.append(m)
        x = jnp.where(x == m, NEG_INF, x)
    return jnp.concatenate(cols, axis=1)


def kernel(x):
    m_rows, n_local = x.shape
    chunk = n_local // N_CHUNKS

    def body(x_ref, out_ref, cand_ref, comm_ref, send_sem, recv_sem):
        i = pl.program_id(0)

        cand_ref[i] = _topk_desc(x_ref[...], K)

        @pl.when(i == N_CHUNKS - 1)
        def _():
            my_x = lax.axis_index("x")
            my_y = lax.axis_index("y")
            my_z = lax.axis_index("z")
            partner = (my_x, my_y, 1 - my_z)

            cands = jnp.concatenate(
                [cand_ref[c] for c in range(N_CHUNKS)], axis=1
            )
            local = _topk_desc(cands, K)
            comm_ref[0] = local

            barrier = pltpu.get_barrier_semaphore()
            pl.semaphore_signal(
                barrier, inc=1,
                device_id=partner, device_id_type=pl.DeviceIdType.MESH,
            )
            pl.semaphore_wait(barrier, 1)

            rdma = pltpu.make_async_remote_copy(
                src_ref=comm_ref.at[0],
                dst_ref=comm_ref.at[1],
                send_sem=send_sem,
                recv_sem=recv_sem,
                device_id=partner,
                device_id_type=pl.DeviceIdType.MESH,
            )
            rdma.start()
            rdma.wait()

            both = jnp.concatenate([local, comm_ref[1]], axis=1)
            out_ref[...] = _topk_desc(both, K)

    return pl.pallas_call(
        body,
        grid=(N_CHUNKS,),
        in_specs=[
            pl.BlockSpec((m_rows, chunk), lambda i: (0, i),
                         memory_space=pltpu.VMEM),
        ],
        out_specs=pl.BlockSpec((m_rows, K), lambda i: (0, 0),
                               memory_space=pltpu.VMEM),
        out_shape=jax.ShapeDtypeStruct((m_rows, K), jnp.float32),
        scratch_shapes=[
            pltpu.VMEM((N_CHUNKS, m_rows, K), jnp.float32),
            pltpu.VMEM((2, m_rows, K), jnp.float32),
            pltpu.SemaphoreType.DMA,
            pltpu.SemaphoreType.DMA,
        ],
        compiler_params=pltpu.CompilerParams(
            collective_id=0,
            vmem_limit_bytes=100 * 1024 * 1024,
        ),
    )(x)


# device time: 135995 ns/iter; 1.4263x vs baseline; 1.4263x over previous
import jax
import jax.numpy as jnp
from jax import lax
from jax.experimental import pallas as pl
from jax.experimental.pallas import tpu as pltpu

K = 32
N_CHUNKS = 8
NEG_INF = float("-inf")


def _topk_desc(x, k):
    cols = []
    for _ in range(k):
        m = jnp.max(x, axis=1, keepdims=True)
        cols.append(m)
        x = jnp.where(x == m, NEG_INF, x)
    return jnp.concatenate(cols, axis=1)


def kernel(x):
    m_rows, n_local = x.shape
    chunk = n_local // N_CHUNKS

    def body(x_ref, out_ref, cand_ref, comm_ref, send_sem, recv_sem):
        i = pl.program_id(0)

        cand_ref[i] = _topk_desc(x_ref[...], K)

        @pl.when(i == N_CHUNKS - 1)
        def _():
            my_x = lax.axis_index("x")
            my_y = lax.axis_index("y")
            my_z = lax.axis_index("z")
            partner = (my_x, my_y, 1 - my_z)

            cands = jnp.concatenate(
                [cand_ref[c] for c in range(N_CHUNKS)], axis=1
            )
            local = _topk_desc(cands, K)
            comm_ref[0] = local

            barrier = pltpu.get_barrier_semaphore()
            pl.semaphore_signal(
                barrier, inc=1,
                device_id=partner, device_id_type=pl.DeviceIdType.MESH,
            )
            pl.semaphore_wait(barrier, 1)

            rdma = pltpu.make_async_remote_copy(
                src_ref=comm_ref.at[0],
                dst_ref=comm_ref.at[1],
                send_sem=send_sem,
                recv_sem=recv_sem,
                device_id=partner,
                device_id_type=pl.DeviceIdType.MESH,
            )
            rdma.start()
            rdma.wait()

            both = jnp.concatenate([local, comm_ref[1]], axis=1)
            out_ref[...] = _topk_desc(both, K)

    return pl.pallas_call(
        body,
        grid=(N_CHUNKS,),
        in_specs=[
            pl.BlockSpec((m_rows, chunk), lambda i: (0, i),
                         memory_space=pltpu.VMEM),
        ],
        out_specs=pl.BlockSpec((m_rows, K), lambda i: (0, 0),
                               memory_space=pltpu.VMEM),
        out_shape=jax.ShapeDtypeStruct((m_rows, K), jnp.float32),
        scratch_shapes=[
            pltpu.VMEM((N_CHUNKS, m_rows, K), jnp.float32),
            pltpu.VMEM((2, m_rows, K), jnp.float32),
            pltpu.SemaphoreType.DMA,
            pltpu.SemaphoreType.DMA,
        ],
        compiler_params=pltpu.CompilerParams(
            collective_id=0,
            vmem_limit_bytes=100 * 1024 * 1024,
        ),
    )(x)


# device time: 135874 ns/iter; 1.4276x vs baseline; 1.0009x over previous
import jax
import jax.numpy as jnp
from jax import lax
from jax.experimental import pallas as pl
from jax.experimental.pallas import tpu as pltpu

K = 32
N_CHUNKS = 8
NEG_INF = float("-inf")


def _topk_desc(x, k):
    cols = []
    for h in range(k):
        m = jnp.max(x, axis=1, keepdims=True)
        cols.append(m)
        if h < k - 1:
            x = jnp.where(x == m, NEG_INF, x)
    return jnp.concatenate(cols, axis=1)


def kernel(x):
    m_rows, n_local = x.shape
    chunk = n_local // N_CHUNKS

    def body(x_ref, out_ref, cand_ref, comm_ref, send_sem, recv_sem):
        i = pl.program_id(0)

        cand_ref[i] = _topk_desc(x_ref[...], K)

        @pl.when(i == N_CHUNKS - 1)
        def _():
            my_x = lax.axis_index("x")
            my_y = lax.axis_index("y")
            my_z = lax.axis_index("z")
            partner = (my_x, my_y, 1 - my_z)

            cands = jnp.concatenate(
                [cand_ref[c] for c in range(N_CHUNKS)], axis=1
            )
            local = _topk_desc(cands, K)
            comm_ref[0] = local

            barrier = pltpu.get_barrier_semaphore()
            pl.semaphore_signal(
                barrier, inc=1,
                device_id=partner, device_id_type=pl.DeviceIdType.MESH,
            )
            pl.semaphore_wait(barrier, 1)

            rdma = pltpu.make_async_remote_copy(
                src_ref=comm_ref.at[0],
                dst_ref=comm_ref.at[1],
                send_sem=send_sem,
                recv_sem=recv_sem,
                device_id=partner,
                device_id_type=pl.DeviceIdType.MESH,
            )
            rdma.start()
            rdma.wait()

            both = jnp.concatenate([local, comm_ref[1]], axis=1)
            out_ref[...] = _topk_desc(both, K)

    return pl.pallas_call(
        body,
        grid=(N_CHUNKS,),
        in_specs=[
            pl.BlockSpec((m_rows, chunk), lambda i: (0, i),
                         memory_space=pltpu.VMEM),
        ],
        out_specs=pl.BlockSpec((m_rows, K), lambda i: (0, 0),
                               memory_space=pltpu.VMEM),
        out_shape=jax.ShapeDtypeStruct((m_rows, K), jnp.float32),
        scratch_shapes=[
            pltpu.VMEM((N_CHUNKS, m_rows, K), jnp.float32),
            pltpu.VMEM((2, m_rows, K), jnp.float32),
            pltpu.SemaphoreType.DMA,
            pltpu.SemaphoreType.DMA,
        ],
        compiler_params=pltpu.CompilerParams(
            collective_id=0,
            vmem_limit_bytes=100 * 1024 * 1024,
        ),
    )(x)
